# Initial kernel scaffold; baseline (speedup 1.0000x reference)
import jax
import jax.numpy as jnp
from jax.experimental import pallas as pl


def kernel(x, Wg, bg, W1, b1, W2, b2):
    raise NotImplementedError("write your pallas kernel here")


# speedup vs baseline: 1.0542x; 1.0542x over previous
import functools

import jax
import jax.numpy as jnp
from jax.experimental import pallas as pl

B, S, H = 4, 2048, 1024
E, K, FF = 8, 2, 4096
N = B * S

NEG_BIG = -1e30


def _gating_body(x_ref, wg_ref, bg_ref, gates_ref, colsum_ref):
    x = x_ref[...]
    logits = jnp.dot(x, wg_ref[...], preferred_element_type=jnp.float32)
    logits = logits + bg_ref[...]
    m = jnp.max(logits, axis=1, keepdims=True)
    ex = jnp.exp(logits - m)
    denom = jnp.sum(ex, axis=1, keepdims=True)
    probs = ex / denom
    colsum_ref[0] = jnp.sum(probs, axis=0, keepdims=True)

    iota = jax.lax.broadcasted_iota(jnp.int32, (1, E), 1)
    p1 = jnp.max(probs, axis=1, keepdims=True)
    i1 = jnp.min(jnp.where(probs == p1, iota, E), axis=1, keepdims=True)
    masked = jnp.where(iota == i1, NEG_BIG, probs)
    p2 = jnp.max(masked, axis=1, keepdims=True)
    i2 = jnp.min(jnp.where(masked == p2, iota, E), axis=1, keepdims=True)
    dsum = p1 + p2
    g1 = p1 / dsum
    g2 = p2 / dsum
    gates_ref[...] = jnp.where(iota == i1, g1, 0.0) + jnp.where(iota == i2, g2, 0.0)


def _moe_body(x_ref, w1_ref, w2_ref, b1_ref, b2_ref, gates_ref, out_ref):
    e = pl.program_id(1)
    f = pl.program_id(2)

    @pl.when((e == 0) & (f == 0))
    def _():
        out_ref[...] = jnp.zeros_like(out_ref)

    iota = jax.lax.broadcasted_iota(jnp.int32, (1, E), 1)
    g = jnp.sum(jnp.where(iota == e, gates_ref[...], 0.0), axis=1, keepdims=True)

    h = jnp.dot(x_ref[...], w1_ref[0], preferred_element_type=jnp.float32)
    h = jnp.maximum(h + b1_ref[0, 0], 0.0).astype(jnp.bfloat16)
    part = jnp.dot(h, w2_ref[0], preferred_element_type=jnp.float32)
    out_ref[...] += g * part

    @pl.when(f == 0)
    def _():
        out_ref[...] += g * b2_ref[0]


def kernel(x, Wg, bg, W1, b1, W2, b2):
    x_flat = x.reshape(-1, H)

    TB = 512
    NT = N // TB
    gates, colsum = pl.pallas_call(
        _gating_body,
        grid=(NT,),
        in_specs=[
            pl.BlockSpec((TB, H), lambda t: (t, 0)),
            pl.BlockSpec((H, E), lambda t: (0, 0)),
            pl.BlockSpec((1, E), lambda t: (0, 0)),
        ],
        out_specs=[
            pl.BlockSpec((TB, E), lambda t: (t, 0)),
            pl.BlockSpec((1, 1, E), lambda t: (t, 0, 0)),
        ],
        out_shape=[
            jax.ShapeDtypeStruct((N, E), jnp.float32),
            jax.ShapeDtypeStruct((NT, 1, E), jnp.float32),
        ],
    )(x_flat, Wg, bg.reshape(1, E))

    counts = jnp.sum(colsum, axis=(0, 1))
    aux = E * jnp.sum((counts / jnp.sum(counts)) * (counts / N))

    TM = 1024
    FB = 2048
    NTM = N // TM
    NF = FF // FB
    x_bf = x_flat.astype(jnp.bfloat16)
    w1_bf = W1.astype(jnp.bfloat16)
    w2_bf = W2.astype(jnp.bfloat16)

    out = pl.pallas_call(
        _moe_body,
        grid=(NTM, E, NF),
        in_specs=[
            pl.BlockSpec((TM, H), lambda t, e, f: (t, 0)),
            pl.BlockSpec((1, H, FB), lambda t, e, f: (e, 0, f)),
            pl.BlockSpec((1, FB, H), lambda t, e, f: (e, f, 0)),
            pl.BlockSpec((1, 1, 1, FB), lambda t, e, f: (e, f, 0, 0)),
            pl.BlockSpec((1, 1, H), lambda t, e, f: (e, 0, 0)),
            pl.BlockSpec((TM, E), lambda t, e, f: (t, 0)),
        ],
        out_specs=pl.BlockSpec((TM, H), lambda t, e, f: (t, 0)),
        out_shape=jax.ShapeDtypeStruct((N, H), jnp.float32),
    )(x_bf, w1_bf, w2_bf, b1.reshape(E, NF, 1, FB), b2.reshape(E, 1, H), gates)

    return out.reshape(x.shape), aux


# speedup vs baseline: 1.6418x; 1.5574x over previous
import jax
import jax.numpy as jnp
from jax import lax
from jax.experimental import pallas as pl
from jax.experimental.pallas import tpu as pltpu
from jax.experimental.pallas import tpu_sc as plsc

B, S, H = 4, 2048, 1024
E, K, FF = 8, 2, 4096
N = B * S
NK = N * K

T = 256
NB = NK // T + E
NP = NB * T

TB = 512
NT = N // TB

NEG_BIG = -1e30

NC = 2
NS = 16
NW = NC * NS
CH = 64
RPW = NP // NW
TW = N // NW


def _gating_body(x_ref, wg_ref, bg_ref, eidx_ref, gates_ref, ranks_ref,
                 colsum_ref, counts_ref, carry_ref):
    t = pl.program_id(0)

    @pl.when(t == 0)
    def _():
        carry_ref[...] = jnp.zeros_like(carry_ref)

    logits = jnp.dot(x_ref[...], wg_ref[...],
                     preferred_element_type=jnp.float32) + bg_ref[...]
    m = jnp.max(logits, axis=1, keepdims=True)
    ex = jnp.exp(logits - m)
    probs = ex / jnp.sum(ex, axis=1, keepdims=True)
    colsum_ref[0] = jnp.sum(probs, axis=0, keepdims=True)

    iota = lax.broadcasted_iota(jnp.int32, (1, E), 1)
    p1 = jnp.max(probs, axis=1, keepdims=True)
    i1 = jnp.min(jnp.where(probs == p1, iota, E), axis=1, keepdims=True)
    masked = jnp.where(iota == i1, NEG_BIG, probs)
    p2 = jnp.max(masked, axis=1, keepdims=True)
    i2 = jnp.min(jnp.where(masked == p2, iota, E), axis=1, keepdims=True)
    dsum = p1 + p2
    gates_ref[...] = jnp.concatenate([p1 / dsum, p2 / dsum], axis=1)
    eidx_ref[...] = jnp.concatenate([i1, i2], axis=1)

    onehot = (iota == i1).astype(jnp.float32) + (iota == i2).astype(jnp.float32)
    r_io = lax.broadcasted_iota(jnp.int32, (TB, TB), 0)
    c_io = lax.broadcasted_iota(jnp.int32, (TB, TB), 1)
    tril = (r_io > c_io).astype(jnp.float32)
    prefix = jnp.dot(tril, onehot, preferred_element_type=jnp.float32)
    base = prefix + carry_ref[...]
    r1 = jnp.sum(jnp.where(iota == i1, base, 0.0), axis=1, keepdims=True)
    r2 = jnp.sum(jnp.where(iota == i2, base, 0.0), axis=1, keepdims=True)
    ranks_ref[...] = jnp.concatenate([r1, r2], axis=1).astype(jnp.int32)

    new_carry = carry_ref[...] + jnp.sum(onehot, axis=0, keepdims=True)
    carry_ref[...] = new_carry
    counts_ref[...] = new_carry


def _dispatch_body(tok_hbm, x_hbm, xg_hbm, idx_v, rows_v, sem):
    wid = lax.axis_index("s") * NC + lax.axis_index("c")
    base = wid * RPW

    def step(i, carry):
        off = base + i * CH
        pltpu.sync_copy(tok_hbm.at[pl.ds(off, CH)], idx_v)
        pltpu.async_copy(x_hbm.at[idx_v], rows_v, sem).wait()
        pltpu.sync_copy(rows_v, xg_hbm.at[pl.ds(off, CH)])
        return carry

    lax.fori_loop(0, RPW // CH, step, 0)


def _ffn_body(be_ref, valid_ref, xg_ref, w1_ref, w2_ref, b1_ref, b2_ref,
              gate_ref, out_ref):
    b = pl.program_id(0)

    @pl.when(valid_ref[b] == 1)
    def _():
        xb = xg_ref[...].astype(jnp.bfloat16)
        h = jnp.dot(xb, w1_ref[0], preferred_element_type=jnp.float32)
        h = jnp.maximum(h + b1_ref[0, 0], 0.0).astype(jnp.bfloat16)
        y = jnp.dot(h, w2_ref[0], preferred_element_type=jnp.float32)
        out_ref[...] = gate_ref[0] * (y + b2_ref[0])

    @pl.when(valid_ref[b] == 0)
    def _():
        out_ref[...] = jnp.zeros_like(out_ref)


def _unperm_body(pos_hbm, yg_hbm, yp0_hbm, yp1_hbm, idx_v, rows_v, sem):
    wid = lax.axis_index("s") * NC + lax.axis_index("c")
    base = wid * TW

    def step(i, carry):
        off = base + i * CH
        pltpu.sync_copy(pos_hbm.at[pl.ds(off, CH)], idx_v)
        pltpu.async_copy(yg_hbm.at[idx_v], rows_v, sem).wait()
        pltpu.sync_copy(rows_v, yp0_hbm.at[pl.ds(off, CH)])
        pltpu.sync_copy(pos_hbm.at[pl.ds(N + off, CH)], idx_v)
        pltpu.async_copy(yg_hbm.at[idx_v], rows_v, sem).wait()
        pltpu.sync_copy(rows_v, yp1_hbm.at[pl.ds(off, CH)])
        return carry

    lax.fori_loop(0, TW // CH, step, 0)


def _combine_body(a_ref, b_ref, o_ref):
    o_ref[...] = a_ref[...] + b_ref[...]


def _sc_mesh():
    return plsc.VectorSubcoreMesh(
        core_axis_name="c", subcore_axis_name="s",
        num_cores=NC, num_subcores=NS)


def kernel(x, Wg, bg, W1, b1, W2, b2):
    x_flat = x.reshape(-1, H)

    eidx, gates2, ranks, colsum, counts = pl.pallas_call(
        _gating_body,
        grid=(NT,),
        in_specs=[
            pl.BlockSpec((TB, H), lambda t: (t, 0)),
            pl.BlockSpec((H, E), lambda t: (0, 0)),
            pl.BlockSpec((1, E), lambda t: (0, 0)),
        ],
        out_specs=[
            pl.BlockSpec((TB, K), lambda t: (t, 0)),
            pl.BlockSpec((TB, K), lambda t: (t, 0)),
            pl.BlockSpec((TB, K), lambda t: (t, 0)),
            pl.BlockSpec((1, 1, E), lambda t: (t, 0, 0)),
            pl.BlockSpec((1, E), lambda t: (0, 0)),
        ],
        out_shape=[
            jax.ShapeDtypeStruct((N, K), jnp.int32),
            jax.ShapeDtypeStruct((N, K), jnp.float32),
            jax.ShapeDtypeStruct((N, K), jnp.int32),
            jax.ShapeDtypeStruct((NT, 1, E), jnp.float32),
            jax.ShapeDtypeStruct((1, E), jnp.float32),
        ],
        scratch_shapes=[pltpu.VMEM((1, E), jnp.float32)],
    )(x_flat, Wg, bg.reshape(1, E))

    cs = jnp.sum(colsum, axis=(0, 1))
    aux = E * jnp.sum((cs / jnp.sum(cs)) * (cs / N))

    counts_i = counts[0].astype(jnp.int32)
    blocks_e = (counts_i + (T - 1)) // T
    cumb = jnp.concatenate(
        [jnp.zeros((1,), jnp.int32), jnp.cumsum(blocks_e, dtype=jnp.int32)])
    bar = jnp.arange(NB, dtype=jnp.int32)
    block_expert = jnp.sum(
        (bar[:, None] >= cumb[None, 1:E]).astype(jnp.int32), axis=1)
    valid = (bar < cumb[E]).astype(jnp.int32)
    row_off = cumb[:E] * T
    pos = jnp.take(row_off, eidx) + ranks
    tok_ids = jnp.broadcast_to(
        jnp.arange(N, dtype=jnp.int32)[:, None], (N, K))
    flat_pos = pos.reshape(-1)
    tok_of_slot = jnp.zeros((NP,), jnp.int32).at[flat_pos].set(
        tok_ids.reshape(-1))
    gate_of_slot = jnp.zeros((NP,), jnp.float32).at[flat_pos].set(
        gates2.reshape(-1))
    posT = pos.T.reshape(-1)

    xg = pl.kernel(
        _dispatch_body,
        out_type=jax.ShapeDtypeStruct((NP, H), jnp.float32),
        mesh=_sc_mesh(),
        scratch_types=[
            pltpu.VMEM((CH,), jnp.int32),
            pltpu.VMEM((CH, H), jnp.float32),
            pltpu.SemaphoreType.DMA,
        ],
    )(tok_of_slot, x_flat)

    w1_bf = W1.astype(jnp.bfloat16)
    w2_bf = W2.astype(jnp.bfloat16)
    yg = pl.pallas_call(
        _ffn_body,
        grid_spec=pltpu.PrefetchScalarGridSpec(
            num_scalar_prefetch=2,
            grid=(NB,),
            in_specs=[
                pl.BlockSpec((T, H), lambda b, be, va: (b, 0)),
                pl.BlockSpec((1, H, FF), lambda b, be, va: (be[b], 0, 0)),
                pl.BlockSpec((1, FF, H), lambda b, be, va: (be[b], 0, 0)),
                pl.BlockSpec((1, 1, FF), lambda b, be, va: (be[b], 0, 0)),
                pl.BlockSpec((1, 1, H), lambda b, be, va: (be[b], 0, 0)),
                pl.BlockSpec((1, T, 1), lambda b, be, va: (b, 0, 0)),
            ],
            out_specs=pl.BlockSpec((T, H), lambda b, be, va: (b, 0)),
        ),
        out_shape=jax.ShapeDtypeStruct((NP, H), jnp.float32),
    )(block_expert, valid, xg, w1_bf, w2_bf, b1.reshape(E, 1, FF),
      b2.reshape(E, 1, H), gate_of_slot.reshape(NB, T, 1))

    yp0, yp1 = pl.kernel(
        _unperm_body,
        out_type=[jax.ShapeDtypeStruct((N, H), jnp.float32)] * 2,
        mesh=_sc_mesh(),
        scratch_types=[
            pltpu.VMEM((CH,), jnp.int32),
            pltpu.VMEM((CH, H), jnp.float32),
            pltpu.SemaphoreType.DMA,
        ],
    )(posT, yg)

    out = pl.pallas_call(
        _combine_body,
        grid=(N // 1024,),
        in_specs=[pl.BlockSpec((1024, H), lambda i: (i, 0))] * 2,
        out_specs=pl.BlockSpec((1024, H), lambda i: (i, 0)),
        out_shape=jax.ShapeDtypeStruct((N, H), jnp.float32),
    )(yp0, yp1)

    return out.reshape(x.shape), aux
